# Initial kernel scaffold; baseline (speedup 1.0000x reference)
import jax
import jax.numpy as jnp
from jax.experimental import pallas as pl


def kernel(questions_index, next_questions_index, input_skills_embedding, next_skills_embedding, input_questions_embedding, next_questions_embedding, input_answers_embedding, feature_embedding, hist_neighbor_index, batch_size, question_neighbors, W_in, b_in, W_next, b_next, W_fa, b_fa, Wx, Wh, b_lstm, att1_w, att1_b, att2_w, att2_b):
    raise NotImplementedError("write your pallas kernel here")


# speedup vs baseline: 8.4020x; 8.4020x over previous
import functools

import jax
import jax.numpy as jnp
from jax import lax
from jax.experimental import pallas as pl
from jax.experimental.pallas import tpu as pltpu
from jax.experimental.pallas import tpu_sc as plsc

E = 128
H = 128
NEXT_N = 2
HIST_K = 4


_NC = 2
_NS = 16
_NW = _NC * _NS
_CHUNK = 112


def _sc_gather(idx_flat, question_neighbors, feature_embedding):
    BT = idx_flat.shape[0]
    per_w = BT // _NW
    n_chunks = per_w // _CHUNK
    assert per_w % _CHUNK == 0 and BT % _NW == 0

    mesh = plsc.VectorSubcoreMesh(core_axis_name="c", subcore_axis_name="s")

    @functools.partial(
        pl.kernel,
        mesh=mesh,
        out_type=[
            jax.ShapeDtypeStruct((BT, E), jnp.float32),
            jax.ShapeDtypeStruct((BT, E), jnp.float32),
        ],
        scratch_types=[
            pltpu.VMEM((_CHUNK,), jnp.int32),
            pltpu.VMEM((_CHUNK,), jnp.int32),
            pltpu.VMEM((_CHUNK,), jnp.int32),
            pltpu.VMEM((_CHUNK,), jnp.int32),
            pltpu.VMEM((_CHUNK,), jnp.int32),
            pltpu.VMEM((_CHUNK, E), jnp.float32),
            pltpu.VMEM((_CHUNK, E), jnp.float32),
            pltpu.SemaphoreType.DMA,
            pltpu.SemaphoreType.DMA,
        ],
    )
    def k(idx_hbm, qnf_hbm, feat_hbm, out0_hbm, out1_hbm,
          qidx_v, g0_v, g1_v, nbr0_v, nbr1_v, rows0_v, rows1_v, sem0, sem1):
        wid = lax.axis_index("s") * _NC + lax.axis_index("c")

        def chunk_body(ci, _):
            base = wid * per_w + ci * _CHUNK
            pltpu.sync_copy(idx_hbm.at[pl.ds(base, _CHUNK)], qidx_v)
            for i in range(_CHUNK // 16):
                q = qidx_v[pl.ds(i * 16, 16)]
                g0_v[pl.ds(i * 16, 16)] = q * 4
                g1_v[pl.ds(i * 16, 16)] = q * 4 + 1
            cp0 = pltpu.async_copy(qnf_hbm.at[g0_v], nbr0_v, sem0)
            cp1 = pltpu.async_copy(qnf_hbm.at[g1_v], nbr1_v, sem1)
            cp0.wait()
            cp1.wait()
            cp0 = pltpu.async_copy(feat_hbm.at[nbr0_v], rows0_v, sem0)
            cp1 = pltpu.async_copy(feat_hbm.at[nbr1_v], rows1_v, sem1)
            cp0.wait()
            cp1.wait()
            pltpu.sync_copy(rows0_v, out0_hbm.at[pl.ds(base, _CHUNK)])
            pltpu.sync_copy(rows1_v, out1_hbm.at[pl.ds(base, _CHUNK)])
            return 0

        lax.fori_loop(0, n_chunks, chunk_body, 0)

    return k(idx_flat, question_neighbors.reshape(-1), feature_embedding)


def _lstm_body(iq_ref, ia_ref, isk_ref, wfa_ref, bfa_ref, wx_ref, wh_ref,
               bl_ref, os_ref, h_ref, c_ref):
    t = pl.program_id(0)

    @pl.when(t == 0)
    def _():
        h_ref[...] = jnp.zeros_like(h_ref)
        c_ref[...] = jnp.zeros_like(c_ref)

    iq = iq_ref[...]
    ia = ia_ref[...]
    isk = isk_ref[...]
    x = (iq @ wfa_ref[0:E, :] + ia @ wfa_ref[E:2 * E, :]
         + isk @ wfa_ref[2 * E:3 * E, :] + bfa_ref[...])
    g = x @ wx_ref[...] + h_ref[...] @ wh_ref[...] + bl_ref[...]
    i = jax.nn.sigmoid(g[:, 0:H])
    f = jax.nn.sigmoid(g[:, H:2 * H])
    gg = jnp.tanh(g[:, 2 * H:3 * H])
    o = jax.nn.sigmoid(g[:, 3 * H:4 * H])
    c2 = f * c_ref[...] + i * gg
    h2 = o * jnp.tanh(c2)
    h_ref[...] = h2
    c_ref[...] = c2
    os_ref[...] = h2[:, None, :]


def _lstm(iq, ia, isk, W_fa, b_fa, Wx, Wh, b_lstm, interpret=False):
    B, T, _ = iq.shape
    emb_spec = pl.BlockSpec((B, 1, E), lambda t: (0, t, 0))
    return pl.pallas_call(
        _lstm_body,
        grid=(T,),
        in_specs=[
            emb_spec, emb_spec, emb_spec,
            pl.BlockSpec((3 * E, H), lambda t: (0, 0)),
            pl.BlockSpec((1, H), lambda t: (0, 0)),
            pl.BlockSpec((H, 4 * H), lambda t: (0, 0)),
            pl.BlockSpec((H, 4 * H), lambda t: (0, 0)),
            pl.BlockSpec((1, 4 * H), lambda t: (0, 0)),
        ],
        out_specs=pl.BlockSpec((B, 1, H), lambda t: (0, t, 0)),
        out_shape=jax.ShapeDtypeStruct((B, T, H), jnp.float32),
        scratch_shapes=[
            pltpu.VMEM((B, H), jnp.float32),
            pltpu.VMEM((B, H), jnp.float32),
        ],
        compiler_params=pltpu.CompilerParams(
            dimension_semantics=("arbitrary",)),
        interpret=interpret,
    )(iq, ia, isk, W_fa, b_fa.reshape(1, H), Wx, Wh, b_lstm.reshape(1, 4 * H))


def _attn_body(nsk_ref, isk_ref, nq_ref, n0_ref, n1_ref, os_ref, wnext_ref,
               bnext_ref, w1_ref, b1_ref, w2_ref, b2_ref, out_ref):
    Bb, T, _ = nsk_ref.shape
    nsk = nsk_ref[...]
    isk = isk_ref[...]
    os = os_ref[...]

    sim = lax.dot_general(nsk, isk, (((2,), (2,)), ((0,), (0,))))
    iota_t = lax.broadcasted_iota(jnp.int32, (Bb, T, T), 1)
    iota_s = lax.broadcasted_iota(jnp.int32, (Bb, T, T), 2)
    work = jnp.where(iota_s <= iota_t, sim, -1e9)

    hists = []
    for _k in range(HIST_K):
        m = jnp.max(work, axis=-1, keepdims=True)
        cand = work == m
        first = jnp.min(jnp.where(cand, iota_s, T), axis=-1, keepdims=True)
        sel = iota_s == first
        hists.append(
            lax.dot_general(sel.astype(jnp.float32), os,
                            (((2,), (1,)), ((0,), (0,)))))
        work = jnp.where(sel, -jnp.inf, work)

    def lin_relu(ref):
        v = ref[...].reshape(Bb * T, E)
        return jax.nn.relu(v @ wnext_ref[...] + bnext_ref[...]).reshape(
            Bb, T, H)

    nn_rows = [lin_relu(nq_ref), lin_relu(n0_ref), lin_relu(n1_ref)]
    nh_rows = [os] + hists

    w1 = w1_ref[...].reshape(1, 1, H)
    w2 = w2_ref[...].reshape(1, 1, H)
    a1 = [jnp.sum(nh * w1, axis=-1) + b1_ref[0, 0] for nh in nh_rows]
    a2 = [jnp.sum(nn * w2, axis=-1) + b2_ref[0, 0] for nn in nn_rows]

    scores = [[jnp.sum(nh * nn, axis=-1) for nn in nn_rows] for nh in nh_rows]
    pair = [[a1[i] + a2[j] for j in range(3)] for i in range(5)]

    mx = pair[0][0]
    for i in range(5):
        for j in range(3):
            mx = jnp.maximum(mx, pair[i][j])
    z = jnp.zeros_like(mx)
    acc = jnp.zeros_like(mx)
    for i in range(5):
        for j in range(3):
            e = jnp.exp(pair[i][j] - mx)
            z = z + e
            acc = acc + e * scores[i][j]
    out_ref[...] = acc / z


def _attn(nsk, isk, nq, n0, n1, os, W_next, b_next, att1_w, att1_b, att2_w,
          att2_b, block_b=32, interpret=False):
    B, T, _ = nsk.shape
    emb_spec = pl.BlockSpec((block_b, T, E), lambda i: (i, 0, 0))
    const2 = lambda shape: pl.BlockSpec(shape, lambda i: (0, 0))
    return pl.pallas_call(
        _attn_body,
        grid=(B // block_b,),
        in_specs=[emb_spec] * 6 + [
            const2((E, H)), const2((1, H)),
            const2((1, H)), const2((1, 1)),
            const2((1, H)), const2((1, 1)),
        ],
        out_specs=pl.BlockSpec((block_b, T), lambda i: (i, 0)),
        out_shape=jax.ShapeDtypeStruct((B, T), jnp.float32),
        compiler_params=pltpu.CompilerParams(
            dimension_semantics=("arbitrary",)),
        interpret=interpret,
    )(nsk, isk, nq, n0, n1, os,
      W_next, b_next.reshape(1, H),
      att1_w.reshape(1, H), att1_b.reshape(1, 1),
      att2_w.reshape(1, H), att2_b.reshape(1, 1))


def kernel(questions_index, next_questions_index, input_skills_embedding,
           next_skills_embedding, input_questions_embedding,
           next_questions_embedding, input_answers_embedding,
           feature_embedding, hist_neighbor_index, batch_size,
           question_neighbors, W_in, b_in, W_next, b_next, W_fa, b_fa,
           Wx, Wh, b_lstm, att1_w, att1_b, att2_w, att2_b):
    B, T = next_questions_index.shape
    idx_flat = next_questions_index.reshape(B * T).astype(jnp.int32)
    n0f, n1f = _sc_gather(idx_flat, question_neighbors.astype(jnp.int32),
                          feature_embedding)
    n0 = n0f.reshape(B, T, E)
    n1 = n1f.reshape(B, T, E)

    os = _lstm(input_questions_embedding, input_answers_embedding,
               input_skills_embedding, W_fa, b_fa, Wx, Wh, b_lstm)

    return _attn(next_skills_embedding, input_skills_embedding,
                 next_questions_embedding, n0, n1, os,
                 W_next, b_next, att1_w, att1_b, att2_w, att2_b)


# speedup vs baseline: 13.8832x; 1.6524x over previous
import functools

import jax
import jax.numpy as jnp
from jax import lax
from jax.experimental import pallas as pl
from jax.experimental.pallas import tpu as pltpu
from jax.experimental.pallas import tpu_sc as plsc

E = 128
H = 128
NEXT_N = 2
HIST_K = 4


_NC = 2
_NS = 16
_NW = _NC * _NS
_CHUNK = 112


def _sc_gather(idx_flat, question_neighbors, feature_embedding):
    BT = idx_flat.shape[0]
    per_w = BT // _NW
    n_chunks = per_w // _CHUNK
    assert per_w % _CHUNK == 0 and BT % _NW == 0

    mesh = plsc.VectorSubcoreMesh(core_axis_name="c", subcore_axis_name="s")

    @functools.partial(
        pl.kernel,
        mesh=mesh,
        out_type=[
            jax.ShapeDtypeStruct((BT, E), jnp.float32),
            jax.ShapeDtypeStruct((BT, E), jnp.float32),
        ],
        scratch_types=[
            pltpu.VMEM((_CHUNK,), jnp.int32),
            pltpu.VMEM((_CHUNK,), jnp.int32),
            pltpu.VMEM((_CHUNK,), jnp.int32),
            pltpu.VMEM((_CHUNK,), jnp.int32),
            pltpu.VMEM((_CHUNK,), jnp.int32),
            pltpu.VMEM((_CHUNK, E), jnp.float32),
            pltpu.VMEM((_CHUNK, E), jnp.float32),
            pltpu.SemaphoreType.DMA,
            pltpu.SemaphoreType.DMA,
        ],
    )
    def k(idx_hbm, qnf_hbm, feat_hbm, out0_hbm, out1_hbm,
          qidx_v, g0_v, g1_v, nbr0_v, nbr1_v, rows0_v, rows1_v, sem0, sem1):
        wid = lax.axis_index("s") * _NC + lax.axis_index("c")

        def chunk_body(ci, _):
            base = wid * per_w + ci * _CHUNK
            pltpu.sync_copy(idx_hbm.at[pl.ds(base, _CHUNK)], qidx_v)
            for i in range(_CHUNK // 16):
                q = qidx_v[pl.ds(i * 16, 16)]
                g0_v[pl.ds(i * 16, 16)] = q * 4
                g1_v[pl.ds(i * 16, 16)] = q * 4 + 1
            cp0 = pltpu.async_copy(qnf_hbm.at[g0_v], nbr0_v, sem0)
            cp1 = pltpu.async_copy(qnf_hbm.at[g1_v], nbr1_v, sem1)
            cp0.wait()
            cp1.wait()
            cp0 = pltpu.async_copy(feat_hbm.at[nbr0_v], rows0_v, sem0)
            cp1 = pltpu.async_copy(feat_hbm.at[nbr1_v], rows1_v, sem1)
            cp0.wait()
            cp1.wait()
            pltpu.sync_copy(rows0_v, out0_hbm.at[pl.ds(base, _CHUNK)])
            pltpu.sync_copy(rows1_v, out1_hbm.at[pl.ds(base, _CHUNK)])
            return 0

        lax.fori_loop(0, n_chunks, chunk_body, 0)

    return k(idx_flat, question_neighbors.reshape(-1), feature_embedding)


def _lstm_body(iq_ref, ia_ref, isk_ref, wfa_ref, bfa_ref, wx_ref, wh_ref,
               bl_ref, os_ref, h_ref, c_ref):
    g_i = pl.program_id(0)

    @pl.when(g_i == 0)
    def _():
        h_ref[...] = jnp.zeros_like(h_ref)
        c_ref[...] = jnp.zeros_like(c_ref)

    h = h_ref[...]
    c = c_ref[...]
    wx = wx_ref[...]
    wh = wh_ref[...]
    for k in range(8):
        x = (iq_ref[:, k, :] @ wfa_ref[0:E, :]
             + ia_ref[:, k, :] @ wfa_ref[E:2 * E, :]
             + isk_ref[:, k, :] @ wfa_ref[2 * E:3 * E, :] + bfa_ref[...])
        g = x @ wx + h @ wh + bl_ref[...]
        i = 0.5 + 0.5 * jnp.tanh(0.5 * g[:, 0:H])
        f = 0.5 + 0.5 * jnp.tanh(0.5 * g[:, H:2 * H])
        gg = jnp.tanh(g[:, 2 * H:3 * H])
        o = 0.5 + 0.5 * jnp.tanh(0.5 * g[:, 3 * H:4 * H])
        c = f * c + i * gg
        h = o * jnp.tanh(c)
        os_ref[:, k, :] = h
    h_ref[...] = h
    c_ref[...] = c


def _lstm(iq, ia, isk, W_fa, b_fa, Wx, Wh, b_lstm, interpret=False):
    B, T, _ = iq.shape
    emb_spec = pl.BlockSpec((B, 8, E), lambda t: (0, t, 0))
    return pl.pallas_call(
        _lstm_body,
        grid=((T + 7) // 8,),
        in_specs=[
            emb_spec, emb_spec, emb_spec,
            pl.BlockSpec((3 * E, H), lambda t: (0, 0)),
            pl.BlockSpec((1, H), lambda t: (0, 0)),
            pl.BlockSpec((H, 4 * H), lambda t: (0, 0)),
            pl.BlockSpec((H, 4 * H), lambda t: (0, 0)),
            pl.BlockSpec((1, 4 * H), lambda t: (0, 0)),
        ],
        out_specs=pl.BlockSpec((B, 8, H), lambda t: (0, t, 0)),
        out_shape=jax.ShapeDtypeStruct((B, T, H), jnp.float32),
        scratch_shapes=[
            pltpu.VMEM((B, H), jnp.float32),
            pltpu.VMEM((B, H), jnp.float32),
        ],
        compiler_params=pltpu.CompilerParams(
            dimension_semantics=("arbitrary",)),
        interpret=interpret,
    )(iq, ia, isk, W_fa, b_fa.reshape(1, H), Wx, Wh, b_lstm.reshape(1, 4 * H))


def _attn_body(nsk_ref, isk_ref, nq_ref, n0_ref, n1_ref, os_ref, wnext_ref,
               bnext_ref, w1_ref, b1_ref, w2_ref, b2_ref, out_ref):
    Bb, T, _ = nsk_ref.shape
    nsk = nsk_ref[...]
    isk = isk_ref[...]
    os = os_ref[...]

    sim = lax.dot_general(nsk, isk, (((2,), (2,)), ((0,), (0,))))
    iota_t = lax.broadcasted_iota(jnp.int32, (Bb, T, T), 1)
    iota_s = lax.broadcasted_iota(jnp.int32, (Bb, T, T), 2)
    work = jnp.where(iota_s <= iota_t, sim, -1e9)

    M = (iota_t == iota_s).astype(jnp.float32)
    for _k in range(HIST_K):
        m = jnp.max(work, axis=-1, keepdims=True)
        cand = work == m
        first = jnp.min(jnp.where(cand, iota_s, T), axis=-1, keepdims=True)
        sel = iota_s == first
        M = M + sel.astype(jnp.float32)
        work = jnp.where(sel, -jnp.inf, work)

    wn = wnext_ref[...]
    bn = bnext_ref[...]
    nn0 = jax.nn.relu(
        lax.dot_general(nq_ref[...], wn, (((2,), (0,)), ((), ())))
        + bn[None])
    nn1 = jax.nn.relu(n0_ref[...] @ wn + bn)
    nn2 = jax.nn.relu(n1_ref[...] @ wn + bn)
    nns = [nn0] + [v.reshape(Bb, T, H) for v in (nn1, nn2)]

    w2b = jnp.broadcast_to(w2_ref[...].reshape(1, 1, H), (Bb, 1, H))
    osA = jnp.concatenate([os, w2b], axis=1)
    SA = [lax.dot_general(nn, osA, (((2,), (2,)), ((0,), (0,))))
          for nn in nns]

    w1b = jnp.broadcast_to(w1_ref[...].reshape(1, 1, H), (Bb, 1, H))
    aosA = lax.dot_general(w1b, osA, (((2,), (2,)), ((0,), (0,))))
    aos = aosA[:, :, :T]

    mx1 = jnp.max(aos, axis=-1, keepdims=True)
    pexp = jnp.exp(aos - mx1)
    a2 = [sa[:, :, T:T + 1] for sa in SA]
    mx2 = jnp.maximum(jnp.maximum(a2[0], a2[1]), a2[2])
    q = [jnp.exp(v - mx2) for v in a2]
    sq = q[0] * SA[0][:, :, :T] + q[1] * SA[1][:, :, :T] \
        + q[2] * SA[2][:, :, :T]
    mp = M * pexp
    num = jnp.sum(mp * sq, axis=-1, keepdims=True)
    psum = jnp.sum(mp, axis=-1, keepdims=True)
    qsum = q[0] + q[1] + q[2]
    out_ref[...] = (num / (psum * qsum)).reshape(Bb, T)


def _attn(nsk, isk, nq, n0f, n1f, os, W_next, b_next, att1_w, att1_b, att2_w,
          att2_b, block_b=64, interpret=False):
    B, T, _ = nsk.shape
    emb_spec = pl.BlockSpec((block_b, T, E), lambda i: (i, 0, 0))
    flat_spec = pl.BlockSpec((block_b * T, E), lambda i: (i, 0))
    const2 = lambda shape: pl.BlockSpec(shape, lambda i: (0, 0))
    return pl.pallas_call(
        _attn_body,
        grid=(B // block_b,),
        in_specs=[emb_spec, emb_spec, emb_spec, flat_spec, flat_spec,
                  emb_spec] + [
            const2((E, H)), const2((1, H)),
            const2((1, H)), const2((1, 1)),
            const2((1, H)), const2((1, 1)),
        ],
        out_specs=pl.BlockSpec((block_b, T), lambda i: (i, 0)),
        out_shape=jax.ShapeDtypeStruct((B, T), jnp.float32),
        compiler_params=pltpu.CompilerParams(
            dimension_semantics=("arbitrary",)),
        interpret=interpret,
    )(nsk, isk, nq, n0f, n1f, os,
      W_next, b_next.reshape(1, H),
      att1_w.reshape(1, H), att1_b.reshape(1, 1),
      att2_w.reshape(1, H), att2_b.reshape(1, 1))


def kernel(questions_index, next_questions_index, input_skills_embedding,
           next_skills_embedding, input_questions_embedding,
           next_questions_embedding, input_answers_embedding,
           feature_embedding, hist_neighbor_index, batch_size,
           question_neighbors, W_in, b_in, W_next, b_next, W_fa, b_fa,
           Wx, Wh, b_lstm, att1_w, att1_b, att2_w, att2_b):
    B, T = next_questions_index.shape
    idx_flat = next_questions_index.reshape(B * T).astype(jnp.int32)
    n0f, n1f = _sc_gather(idx_flat, question_neighbors.astype(jnp.int32),
                          feature_embedding)

    os = _lstm(input_questions_embedding, input_answers_embedding,
               input_skills_embedding, W_fa, b_fa, Wx, Wh, b_lstm)

    return _attn(next_skills_embedding, input_skills_embedding,
                 next_questions_embedding, n0f, n1f, os,
                 W_next, b_next, att1_w, att1_b, att2_w, att2_b)
